# Initial kernel scaffold; baseline (speedup 1.0000x reference)
import functools

import jax
import jax.numpy as jnp
from jax import lax
from jax.experimental import pallas as pl
from jax.experimental.pallas import tpu as pltpu
from jax.experimental.pallas import tpu_sc as plsc

B = 16384
ITEM_D = 32
CAT_D = 6
BRAND_D = 10
OUT_D = ITEM_D + CAT_D + BRAND_D

NC = 2
NS = 16
NW = NC * NS
BPW = B // NW
CHUNK = 128
NCHUNK = BPW // CHUNK


def _body(item_idx_hbm, cat_idx_hbm, brand_idx_hbm,
          item_tbl_hbm, cat_tbl_hbm, brand_tbl_hbm,
          out_hbm,
          ii_v, ic_v, ib_v, item_rows, cat_rows, brand_rows, sem):
    wid = lax.axis_index("s") * NC + lax.axis_index("c")
    base = wid * NCHUNK

    pltpu.sync_copy(item_idx_hbm.at[pl.ds(base, NCHUNK)], ii_v)
    pltpu.sync_copy(cat_idx_hbm.at[pl.ds(base, NCHUNK)], ic_v)
    pltpu.sync_copy(brand_idx_hbm.at[pl.ds(base, NCHUNK)], ib_v)

    for ref in (ii_v, ic_v, ib_v):
        for j in range(NCHUNK):
            for i in range(CHUNK // 16):
                sl = pl.ds(i * 16, 16)
                ref[j, sl] = ref[j, sl] + 1

    copies = []
    for j in range(NCHUNK):
        dst = pl.ds(j * CHUNK, CHUNK)
        copies.append(pltpu.make_async_copy(
            item_tbl_hbm.at[ii_v.at[j]], item_rows.at[dst], sem))
        copies.append(pltpu.make_async_copy(
            cat_tbl_hbm.at[ic_v.at[j]], cat_rows.at[dst], sem))
        copies.append(pltpu.make_async_copy(
            brand_tbl_hbm.at[ib_v.at[j]], brand_rows.at[dst], sem))
    for c in copies:
        c.start()
    for c in copies:
        c.wait()

    rows = pl.ds(wid * BPW, BPW)
    pltpu.sync_copy(item_rows, out_hbm.at[rows, pl.ds(0, ITEM_D)])
    pltpu.sync_copy(cat_rows, out_hbm.at[rows, pl.ds(ITEM_D, CAT_D)])
    pltpu.sync_copy(brand_rows, out_hbm.at[rows, pl.ds(ITEM_D + CAT_D, BRAND_D)])


def kernel(item_id, category_id, brand_id, item_table, cat_table, brand_table):
    mesh = plsc.VectorSubcoreMesh(core_axis_name="c", subcore_axis_name="s")
    k = functools.partial(
        pl.kernel,
        mesh=mesh,
        out_type=jax.ShapeDtypeStruct((B, OUT_D), jnp.float32),
        scratch_types=[
            pltpu.VMEM((NCHUNK, CHUNK), jnp.int32),
            pltpu.VMEM((NCHUNK, CHUNK), jnp.int32),
            pltpu.VMEM((NCHUNK, CHUNK), jnp.int32),
            pltpu.VMEM((BPW, ITEM_D), jnp.float32),
            pltpu.VMEM((BPW, CAT_D), jnp.float32),
            pltpu.VMEM((BPW, BRAND_D), jnp.float32),
            pltpu.SemaphoreType.DMA,
        ],
    )(_body)
    return k(item_id.reshape(B // CHUNK, CHUNK),
             category_id.reshape(B // CHUNK, CHUNK),
             brand_id.reshape(B // CHUNK, CHUNK),
             item_table, cat_table, brand_table)


# speedup vs baseline: 4.9496x; 4.9496x over previous
import functools

import jax
import jax.numpy as jnp
from jax import lax
from jax.experimental import pallas as pl
from jax.experimental.pallas import tpu as pltpu
from jax.experimental.pallas import tpu_sc as plsc

B = 16384
ITEM_D = 32
CAT_D = 6
BRAND_D = 10
TAIL_D = CAT_D + BRAND_D
OUT_D = ITEM_D + TAIL_D

CAT_WORDS = 101 * CAT_D
CAT_PAD = 624
BRAND_WORDS = 201 * BRAND_D
BRAND_PAD = 2016

NC = 2
NS = 16
NW = NC * NS
BPW = B // NW
CHUNK = 128
NCHUNK = BPW // CHUNK
NGROUP = BPW // 16


def _body(item_idx_hbm, cat_idx_hbm, brand_idx_hbm,
          item_tbl_hbm, cat_flat_hbm, brand_flat_hbm,
          out_hbm,
          ii_v, ic_v, ib_v, ig, tb, cat_vm, brand_vm, sem):
    wid = lax.axis_index("s") * NC + lax.axis_index("c")
    base = wid * NCHUNK

    pltpu.sync_copy(item_idx_hbm.at[pl.ds(base, NCHUNK)], ii_v)
    pltpu.sync_copy(cat_idx_hbm.at[pl.ds(base, NCHUNK)], ic_v)
    pltpu.sync_copy(brand_idx_hbm.at[pl.ds(base, NCHUNK)], ib_v)
    pltpu.sync_copy(cat_flat_hbm, cat_vm)
    pltpu.sync_copy(brand_flat_hbm, brand_vm)

    for j in range(NCHUNK):
        for i in range(CHUNK // 16):
            sl = pl.ds(i * 16, 16)
            ii_v[j, sl] = ii_v[j, sl] + 1

    copies = [
        pltpu.make_async_copy(
            item_tbl_hbm.at[ii_v.at[j]],
            ig.at[pl.ds(j * CHUNK, CHUNK)], sem)
        for j in range(NCHUNK)
    ]
    for c in copies:
        c.start()

    iota = lax.iota(jnp.int32, 16)
    for g in range(NGROUP):
        j, i = divmod(g, CHUNK // 16)
        sl = pl.ds(i * 16, 16)
        rows = iota + (g * 16)
        ic6 = ic_v[j, sl] * CAT_D
        ib10 = ib_v[j, sl] * BRAND_D
        for c in range(TAIL_D):
            if c < CAT_D:
                v = plsc.load_gather(cat_vm, [ic6 + (CAT_D + c)])
            else:
                v = plsc.load_gather(brand_vm, [ib10 + (BRAND_D + c - CAT_D)])
            plsc.store_scatter(tb, [rows, jnp.full((16,), c, jnp.int32)], v)

    for c in copies:
        c.wait()

    rows = pl.ds(wid * BPW, BPW)
    pltpu.sync_copy(ig, out_hbm.at[rows, pl.ds(0, ITEM_D)])
    pltpu.sync_copy(tb, out_hbm.at[rows, pl.ds(ITEM_D, TAIL_D)])


def kernel(item_id, category_id, brand_id, item_table, cat_table, brand_table):
    mesh = plsc.VectorSubcoreMesh(core_axis_name="c", subcore_axis_name="s")
    k = functools.partial(
        pl.kernel,
        mesh=mesh,
        compiler_params=pltpu.CompilerParams(use_tc_tiling_on_sc=False,
                                             needs_layout_passes=False),
        out_type=jax.ShapeDtypeStruct((B, OUT_D), jnp.float32),
        scratch_types=[
            pltpu.VMEM((NCHUNK, CHUNK), jnp.int32),
            pltpu.VMEM((NCHUNK, CHUNK), jnp.int32),
            pltpu.VMEM((NCHUNK, CHUNK), jnp.int32),
            pltpu.VMEM((BPW, ITEM_D), jnp.float32),
            pltpu.VMEM((BPW, TAIL_D), jnp.float32),
            pltpu.VMEM((CAT_PAD,), jnp.float32),
            pltpu.VMEM((BRAND_PAD,), jnp.float32),
            pltpu.SemaphoreType.DMA,
        ],
    )(_body)
    cat_flat = jnp.zeros((CAT_PAD,), jnp.float32).at[:CAT_WORDS].set(
        cat_table.reshape(-1))
    brand_flat = jnp.zeros((BRAND_PAD,), jnp.float32).at[:BRAND_WORDS].set(
        brand_table.reshape(-1))
    return k(item_id.reshape(B // CHUNK, CHUNK),
             category_id.reshape(B // CHUNK, CHUNK),
             brand_id.reshape(B // CHUNK, CHUNK),
             item_table, cat_flat, brand_flat)
